# baseline (speedup 1.0000x reference)
import functools

import jax
import jax.numpy as jnp
from jax import lax
from jax.experimental import pallas as pl
from jax.experimental.pallas import tpu as pltpu
from jax.experimental.pallas import tpu_sc as plsc

B, H, W, C = 32, 32, 32, 768
E = 64
TOP_K = 8
PH = PW = 2
NPH, NPW = H // PH, W // PW
P = NPH * NPW

NSC = 8
NTC = B - NSC
NSTREAM = 8
IMGS = NSTREAM // 2
HH = H // 2
R = P * IMGS

NW = 32
NPR = NSC * NPH // NW


def _route(means, gw_bf, w_ref, i_ref, l_ref):
    n = means.shape[0]
    logits = jax.lax.dot_general(
        means.astype(jnp.bfloat16), gw_bf,
        dimension_numbers=(((1,), (1,)), ((), ())),
        preferred_element_type=jnp.float32,
    )
    l_ref[...] = logits
    iota_f = jax.lax.broadcasted_iota(jnp.int32, (n, E), 1).astype(jnp.float32)
    vals = logits
    ws, ids = [], []
    neg = jnp.float32(-jnp.inf)
    for _ in range(TOP_K):
        mk = jnp.max(vals, axis=-1, keepdims=True)
        idx = jnp.min(jnp.where(vals >= mk, iota_f, jnp.float32(E)),
                      axis=-1, keepdims=True)
        ws.append(mk)
        ids.append(idx)
        vals = jnp.where(iota_f == idx, neg, vals)
    lcat = jnp.concatenate(ws, axis=-1)
    icat = jnp.concatenate(ids, axis=-1)
    ex = jnp.exp(lcat - lcat[:, :1])
    w_ref[...] = ex / jnp.sum(ex, axis=-1, keepdims=True)
    i_ref[...] = icat.astype(jnp.int32)


def _tc_main_kernel(*refs):
    x_refs = refs[:NSTREAM]
    gw_ref, w_ref, i_ref, l_ref = refs[NSTREAM:]
    gw = gw_ref[...].astype(jnp.bfloat16)
    means_parts = []
    for q in range(NSTREAM):
        xb = x_refs[q][0]
        s = xb[:, :, :C] + xb[:, :, C:]
        s = s.reshape(HH // 2, 2, NPW, C)
        s = s[:, 0, :, :] + s[:, 1, :, :]
        means_parts.append(s.reshape(P // 2, C))
    means = jnp.concatenate(means_parts, axis=0) * 0.25
    _route(means, gw, w_ref, i_ref, l_ref)


def _tc_tail_kernel(m_ref, gw_ref, w_ref, i_ref, l_ref):
    _route(m_ref[...], gw_ref[...].astype(jnp.bfloat16), w_ref, i_ref, l_ref)


def _sc_pool_kernel(x_hbm, out_hbm, buf, obuf):
    wid = lax.axis_index("s") * 2 + lax.axis_index("c")

    def body(k, _):
        t = wid * NPR + k
        row0 = NTC * 512 + t * 32
        pltpu.sync_copy(x_hbm.at[pl.ds(row0, 32)], buf)

        def jloop(j, _):
            def cloop(c, _):
                a = buf[j, pl.ds(c * 16, 16)]
                b2 = buf[j, pl.ds(C + c * 16, 16)]
                c1 = buf[j + NPW, pl.ds(c * 16, 16)]
                d = buf[j + NPW, pl.ds(C + c * 16, 16)]
                obuf[j, pl.ds(c * 16, 16)] = ((a + b2) + (c1 + d)) * 0.25
                return 0
            return lax.fori_loop(0, C // 16, cloop, 0)

        lax.fori_loop(0, NPW, jloop, 0)
        pltpu.sync_copy(obuf, out_hbm.at[pl.ds(t * NPW, NPW)])
        return 0

    lax.fori_loop(0, NPR, body, 0)


@jax.jit
def kernel(x, spatial_shape, gate_w):
    del spatial_shape
    b = x.shape[0]
    xflat = x.reshape(b * 512, 2 * C)

    sc_pool = pl.kernel(
        _sc_pool_kernel,
        out_type=jax.ShapeDtypeStruct((NSC * P, C), jnp.float32),
        mesh=plsc.VectorSubcoreMesh(core_axis_name="c", subcore_axis_name="s"),
        scratch_types=[
            pltpu.VMEM((2 * NPW, 2 * C), jnp.float32),
            pltpu.VMEM((NPW, C), jnp.float32),
        ],
    )
    means_sc = sc_pool(xflat)

    x5 = x.reshape(b * 2, HH, W // 2, 2 * C)

    def make_spec(q):
        return pl.BlockSpec((1, HH, W // 2, 2 * C),
                            lambda i, q=q: (NSTREAM * i + q, 0, 0, 0))

    out_main = pl.pallas_call(
        _tc_main_kernel,
        grid=(NTC // IMGS,),
        in_specs=[make_spec(q) for q in range(NSTREAM)]
        + [pl.BlockSpec((E, C), lambda i: (0, 0))],
        out_specs=[
            pl.BlockSpec((R, TOP_K), lambda i: (i, 0)),
            pl.BlockSpec((R, TOP_K), lambda i: (i, 0)),
            pl.BlockSpec((R, E), lambda i: (i, 0)),
        ],
        out_shape=[
            jax.ShapeDtypeStruct((NTC * P, TOP_K), jnp.float32),
            jax.ShapeDtypeStruct((NTC * P, TOP_K), jnp.int32),
            jax.ShapeDtypeStruct((NTC * P, E), jnp.float32),
        ],
    )(*([x5] * NSTREAM), gate_w)

    out_tail = pl.pallas_call(
        _tc_tail_kernel,
        grid=(1,),
        in_specs=[
            pl.BlockSpec((NSC * P, C), lambda i: (0, 0)),
            pl.BlockSpec((E, C), lambda i: (0, 0)),
        ],
        out_specs=[
            pl.BlockSpec((NSC * P, TOP_K), lambda i: (0, 0)),
            pl.BlockSpec((NSC * P, TOP_K), lambda i: (0, 0)),
            pl.BlockSpec((NSC * P, E), lambda i: (0, 0)),
        ],
        out_shape=[
            jax.ShapeDtypeStruct((NSC * P, TOP_K), jnp.float32),
            jax.ShapeDtypeStruct((NSC * P, TOP_K), jnp.int32),
            jax.ShapeDtypeStruct((NSC * P, E), jnp.float32),
        ],
    )(means_sc, gate_w)

    return (jnp.concatenate([out_main[0], out_tail[0]], axis=0),
            jnp.concatenate([out_main[1], out_tail[1]], axis=0),
            jnp.concatenate([out_main[2], out_tail[2]], axis=0))


# speedup vs baseline: 1.4883x; 1.4883x over previous
import functools

import jax
import jax.numpy as jnp
from jax import lax
from jax.experimental import pallas as pl
from jax.experimental.pallas import tpu as pltpu
from jax.experimental.pallas import tpu_sc as plsc

B, H, W, C = 32, 32, 32, 768
E = 64
TOP_K = 8
PH = PW = 2
NPH, NPW = H // PH, W // PW
P = NPH * NPW

NSC = 8
NTC = B - NSC
NSTREAM = 8
IMGS = NSTREAM // 2
HH = H // 2
R = P * IMGS

NW = 32
NPR = NSC * NPH // NW


def _route(means, gw_bf, w_ref, i_ref, l_ref):
    n = means.shape[0]
    logits = jax.lax.dot_general(
        means.astype(jnp.bfloat16), gw_bf,
        dimension_numbers=(((1,), (1,)), ((), ())),
        preferred_element_type=jnp.float32,
    )
    l_ref[...] = logits
    iota_f = jax.lax.broadcasted_iota(jnp.int32, (n, E), 1).astype(jnp.float32)
    vals = logits
    ws, ids = [], []
    neg = jnp.float32(-jnp.inf)
    for _ in range(TOP_K):
        mk = jnp.max(vals, axis=-1, keepdims=True)
        idx = jnp.min(jnp.where(vals >= mk, iota_f, jnp.float32(E)),
                      axis=-1, keepdims=True)
        ws.append(mk)
        ids.append(idx)
        vals = jnp.where(iota_f == idx, neg, vals)
    lcat = jnp.concatenate(ws, axis=-1)
    icat = jnp.concatenate(ids, axis=-1)
    ex = jnp.exp(lcat - lcat[:, :1])
    w_ref[...] = ex / jnp.sum(ex, axis=-1, keepdims=True)
    i_ref[...] = icat.astype(jnp.int32)


def _tc_main_kernel(*refs):
    x_refs = refs[:NSTREAM]
    gw_ref, w_ref, i_ref, l_ref = refs[NSTREAM:]
    gw = gw_ref[...].astype(jnp.bfloat16)
    means_parts = []
    for q in range(NSTREAM):
        xb = x_refs[q][...]
        s = xb[:, :C] + xb[:, C:]
        s = s.reshape(HH // 2, 2, NPW, C)
        s = s[:, 0, :, :] + s[:, 1, :, :]
        means_parts.append(s.reshape(P // 2, C))
    means = jnp.concatenate(means_parts, axis=0) * 0.25
    _route(means, gw, w_ref, i_ref, l_ref)


def _tc_tail_kernel(m_ref, gw_ref, w_ref, i_ref, l_ref):
    _route(m_ref[...], gw_ref[...].astype(jnp.bfloat16), w_ref, i_ref, l_ref)


def _sc_pool_kernel(x_hbm, out_hbm, buf, obuf):
    wid = lax.axis_index("s") * 2 + lax.axis_index("c")

    def body(k, _):
        t = wid * NPR + k
        row0 = NTC * 512 + t * 32
        pltpu.sync_copy(x_hbm.at[pl.ds(row0, 32)], buf)

        def jloop(j, _):
            def cloop(c, _):
                a = buf[j, pl.ds(c * 16, 16)]
                b2 = buf[j, pl.ds(C + c * 16, 16)]
                c1 = buf[j + NPW, pl.ds(c * 16, 16)]
                d = buf[j + NPW, pl.ds(C + c * 16, 16)]
                obuf[j, pl.ds(c * 16, 16)] = ((a + b2) + (c1 + d)) * 0.25
                return 0
            return lax.fori_loop(0, C // 16, cloop, 0)

        lax.fori_loop(0, NPW, jloop, 0)
        pltpu.sync_copy(obuf, out_hbm.at[pl.ds(t * NPW, NPW)])
        return 0

    lax.fori_loop(0, NPR, body, 0)


@jax.jit
def kernel(x, spatial_shape, gate_w):
    del spatial_shape
    b = x.shape[0]
    xflat = x.reshape(b * 512, 2 * C)

    sc_pool = pl.kernel(
        _sc_pool_kernel,
        out_type=jax.ShapeDtypeStruct((NSC * P, C), jnp.float32),
        mesh=plsc.VectorSubcoreMesh(core_axis_name="c", subcore_axis_name="s"),
        scratch_types=[
            pltpu.VMEM((2 * NPW, 2 * C), jnp.float32),
            pltpu.VMEM((NPW, C), jnp.float32),
        ],
    )
    means_sc = sc_pool(xflat)

    HROWS = HH * (W // 2)

    def make_spec(q):
        return pl.BlockSpec((HROWS, 2 * C),
                            lambda i, q=q: (NSTREAM * i + q, 0))

    out_main = pl.pallas_call(
        _tc_main_kernel,
        grid=(NTC // IMGS,),
        in_specs=[make_spec(q) for q in range(NSTREAM)]
        + [pl.BlockSpec((E, C), lambda i: (0, 0))],
        out_specs=[
            pl.BlockSpec((R, TOP_K), lambda i: (i, 0)),
            pl.BlockSpec((R, TOP_K), lambda i: (i, 0)),
            pl.BlockSpec((R, E), lambda i: (i, 0)),
        ],
        out_shape=[
            jax.ShapeDtypeStruct((NTC * P, TOP_K), jnp.float32),
            jax.ShapeDtypeStruct((NTC * P, TOP_K), jnp.int32),
            jax.ShapeDtypeStruct((NTC * P, E), jnp.float32),
        ],
    )(*([xflat] * NSTREAM), gate_w)

    out_tail = pl.pallas_call(
        _tc_tail_kernel,
        grid=(1,),
        in_specs=[
            pl.BlockSpec((NSC * P, C), lambda i: (0, 0)),
            pl.BlockSpec((E, C), lambda i: (0, 0)),
        ],
        out_specs=[
            pl.BlockSpec((NSC * P, TOP_K), lambda i: (0, 0)),
            pl.BlockSpec((NSC * P, TOP_K), lambda i: (0, 0)),
            pl.BlockSpec((NSC * P, E), lambda i: (0, 0)),
        ],
        out_shape=[
            jax.ShapeDtypeStruct((NSC * P, TOP_K), jnp.float32),
            jax.ShapeDtypeStruct((NSC * P, TOP_K), jnp.int32),
            jax.ShapeDtypeStruct((NSC * P, E), jnp.float32),
        ],
    )(means_sc, gate_w)

    return (jnp.concatenate([out_main[0], out_tail[0]], axis=0),
            jnp.concatenate([out_main[1], out_tail[1]], axis=0),
            jnp.concatenate([out_main[2], out_tail[2]], axis=0))
